# baseline (speedup 1.0000x reference)
import functools

import jax
import jax.numpy as jnp
from jax import lax
from jax.experimental import pallas as pl
from jax.experimental.pallas import tpu as pltpu
from jax.experimental.pallas import tpu_sc as plsc

N = 10000
E = 320000
D = 128
NC = 2
NS = 16
NW = NC * NS
RANGE = 320
NPAD = NW * RANGE
CAP = 12288
SCH = 2000
NSCAN = E // SCH
CH = 128


def _sc_spmm(features, src, dst, vals):
    mesh = plsc.VectorSubcoreMesh(core_axis_name="c", subcore_axis_name="s")

    @functools.partial(
        pl.kernel,
        out_type=jax.ShapeDtypeStruct((NPAD, D), jnp.float32),
        mesh=mesh,
        scratch_types=[
            pltpu.VMEM((SCH,), jnp.int32),
            pltpu.VMEM((SCH,), jnp.int32),
            pltpu.VMEM((SCH,), jnp.float32),
            pltpu.VMEM((CAP,), jnp.int32),
            pltpu.VMEM((CAP,), jnp.int32),
            pltpu.VMEM((CAP,), jnp.float32),
            pltpu.VMEM((CH, D), jnp.float32),
            pltpu.VMEM((RANGE, D), jnp.float32),
            pltpu.SemaphoreType.DMA,
        ],
        compiler_params=pltpu.CompilerParams(needs_layout_passes=False),
    )
    def k(feat_hbm, src_hbm, dst_hbm, vals_hbm, out_hbm,
          dscan, sscan, vscan, dl_list, s_list, v_list, rows, acc, sem):
        c = lax.axis_index("c")
        s = lax.axis_index("s")
        wid = s * NC + c
        lo = wid * RANGE
        lov = jnp.full((16,), lo, jnp.int32)
        hiv = jnp.full((16,), lo + RANGE, jnp.int32)
        iota = lax.iota(jnp.int32, 16)
        zeros16 = jnp.zeros((16,), jnp.float32)

        def zrow(r, _):
            for d in range(D // 16):
                acc[r, pl.ds(d * 16, 16)] = zeros16
            return _
        lax.fori_loop(0, RANGE, zrow, None, unroll=4)

        def scan_chunk(ci, cntv):
            off = ci * SCH
            pltpu.sync_copy(dst_hbm.at[pl.ds(off, SCH)], dscan)
            pltpu.sync_copy(src_hbm.at[pl.ds(off, SCH)], sscan)
            pltpu.sync_copy(vals_hbm.at[pl.ds(off, SCH)], vscan)

            def scan_vec(vi, cntv):
                sl = pl.ds(vi * 16, 16)
                d16 = dscan[sl]
                m = (d16 >= lov) & (d16 < hiv)
                mi = m.astype(jnp.int32)
                pos = plsc.cumsum(mi) - mi + cntv
                pos = jnp.minimum(pos, CAP - 1)
                plsc.store_scatter(dl_list, [pos], d16 - lov, mask=m)
                plsc.store_scatter(s_list, [pos], sscan[sl], mask=m)
                plsc.store_scatter(v_list, [pos], vscan[sl], mask=m)
                return cntv + plsc.all_reduce_population_count(m)

            return lax.fori_loop(0, SCH // 16, scan_vec, cntv, unroll=5)

        cntv = lax.fori_loop(0, NSCAN, scan_chunk,
                             jnp.zeros((16,), jnp.int32))

        for kk in range(CH // 16):
            addr = jnp.minimum(cntv + iota + kk * 16, CAP - 1)
            plsc.store_scatter(dl_list, [addr], jnp.zeros((16,), jnp.int32),
                               mask=None)
            plsc.store_scatter(s_list, [addr], jnp.zeros((16,), jnp.int32),
                               mask=None)
            plsc.store_scatter(v_list, [addr], zeros16, mask=None)

        cnt = jnp.max(cntv)
        nch = (cnt + CH - 1) // CH

        def chunk_body(i, _):
            base = i * CH
            pltpu.async_copy(
                feat_hbm.at[s_list.at[pl.ds(base, CH)]], rows, sem).wait()
            cntb = jnp.full((16,), cnt - base, jnp.int32)

            def edge_body(e, _):
                ev = jnp.full((16,), e, jnp.int32)
                vv = plsc.load_gather(v_list, [ev + base])
                vv = jnp.where(ev < cntb, vv, zeros16)
                dlv = plsc.load_gather(dl_list, [ev + base])
                for d in range(D // 16):
                    csl = pl.ds(d * 16, 16)
                    plsc.addupdate_scatter(
                        acc, [dlv, iota + d * 16], rows[e, csl] * vv)
                return _
            lax.fori_loop(0, CH, edge_body, None)
            return _
        lax.fori_loop(0, nch, chunk_body, None)

        pltpu.sync_copy(acc, out_hbm.at[pl.ds(lo, RANGE)])

    return k(features, src, dst, vals)


def _tc_combine(features, lap, W1, b1, W2, b2):
    BN = 1000
    bias = (b1 + b2).reshape(1, D)

    def body(f_ref, l_ref, w1_ref, w2_ref, b_ref, o_ref):
        lap_b = l_ref[...]
        f = f_ref[...]
        m1 = lap_b + f
        m2 = lap_b * f
        dn = (((1,), (1,)), ((), ()))
        o_ref[...] = (
            lax.dot_general(m1, w1_ref[...], dn,
                            preferred_element_type=jnp.float32)
            + lax.dot_general(m2, w2_ref[...], dn,
                              preferred_element_type=jnp.float32)
            + b_ref[...]
        )

    row_spec = pl.BlockSpec((BN, D), lambda i: (i, 0))
    full_spec = pl.BlockSpec((D, D), lambda i: (0, 0))
    return pl.pallas_call(
        body,
        grid=(N // BN,),
        in_specs=[row_spec, row_spec, full_spec, full_spec,
                  pl.BlockSpec((1, D), lambda i: (0, 0))],
        out_specs=row_spec,
        out_shape=jax.ShapeDtypeStruct((N, D), jnp.float32),
    )(features, lap, W1, W2, bias)


@jax.jit
def kernel(features, edge_index, edge_vals, W1, b1, W2, b2):
    dst = edge_index[0]
    src = edge_index[1]
    lap_pad = _sc_spmm(features, src, dst, edge_vals)
    return _tc_combine(features, lap_pad[:N], W1, b1, W2, b2)


# speedup vs baseline: 1.1963x; 1.1963x over previous
import functools

import jax
import jax.numpy as jnp
from jax import lax
from jax.experimental import pallas as pl
from jax.experimental.pallas import tpu as pltpu
from jax.experimental.pallas import tpu_sc as plsc

N = 10000
E = 320000
D = 128
NC = 2
NS = 16
NW = NC * NS
RANGE = 320
NPAD = NW * RANGE
CAP = 11264
SCH = 2000
NSCAN = E // SCH
CH = 128


def _sc_spmm(features, src, dst, vals):
    mesh = plsc.VectorSubcoreMesh(core_axis_name="c", subcore_axis_name="s")

    @functools.partial(
        pl.kernel,
        out_type=jax.ShapeDtypeStruct((NPAD, D), jnp.float32),
        mesh=mesh,
        scratch_types=[
            pltpu.VMEM((SCH,), jnp.int32),
            pltpu.VMEM((SCH,), jnp.int32),
            pltpu.VMEM((SCH,), jnp.int32),
            pltpu.VMEM((SCH,), jnp.int32),
            pltpu.VMEM((SCH,), jnp.float32),
            pltpu.VMEM((SCH,), jnp.float32),
            pltpu.VMEM((CAP,), jnp.int32),
            pltpu.VMEM((CAP,), jnp.int32),
            pltpu.VMEM((CAP,), jnp.float32),
            pltpu.VMEM((CH, D), jnp.float32),
            pltpu.VMEM((CH, D), jnp.float32),
            pltpu.VMEM((RANGE, D), jnp.float32),
            pltpu.SemaphoreType.DMA,
            pltpu.SemaphoreType.DMA,
        ],
        compiler_params=pltpu.CompilerParams(needs_layout_passes=False),
    )
    def k(feat_hbm, src_hbm, dst_hbm, vals_hbm, out_hbm,
          dscan0, dscan1, sscan0, sscan1, vscan0, vscan1,
          dl_list, s_list, v_list, rows0, rows1, acc, sem0, sem1):
        dscans = (dscan0, dscan1)
        sscans = (sscan0, sscan1)
        vscans = (vscan0, vscan1)
        rowss = (rows0, rows1)
        c = lax.axis_index("c")
        s = lax.axis_index("s")
        wid = s * NC + c
        lo = wid * RANGE
        lov = jnp.full((16,), lo, jnp.int32)
        rngv = jnp.full((16,), RANGE, jnp.uint32)
        iota = lax.iota(jnp.int32, 16)
        zeros16 = jnp.zeros((16,), jnp.float32)
        sems = (sem0, sem1)

        def zrow(r, _):
            for d in range(D // 16):
                acc[r, pl.ds(d * 16, 16)] = zeros16
            return _
        lax.fori_loop(0, RANGE, zrow, None, unroll=4)

        def scan_issue(ci, b):
            off = ci * SCH
            pltpu.async_copy(dst_hbm.at[pl.ds(off, SCH)], dscans[b],
                             sems[b])
            pltpu.async_copy(src_hbm.at[pl.ds(off, SCH)], sscans[b],
                             sems[b])
            pltpu.async_copy(vals_hbm.at[pl.ds(off, SCH)], vscans[b],
                             sems[b])

        def scan_drain(ci, b):
            off = ci * SCH
            pltpu.make_async_copy(dst_hbm.at[pl.ds(off, SCH)], dscans[b],
                                  sems[b]).wait()
            pltpu.make_async_copy(src_hbm.at[pl.ds(off, SCH)], sscans[b],
                                  sems[b]).wait()
            pltpu.make_async_copy(vals_hbm.at[pl.ds(off, SCH)], vscans[b],
                                  sems[b]).wait()

        def scan_compute(b, cntv):
            def scan_vec(vi, cntv):
                sl = pl.ds(vi * 16, 16)
                d16 = dscans[b][sl]
                dl16 = d16 - lov
                m = plsc.bitcast(dl16, jnp.uint32) < rngv
                mi = m.astype(jnp.int32)
                pos = plsc.cumsum(mi) - mi + cntv
                pos = jnp.minimum(pos, CAP - 1)
                plsc.store_scatter(dl_list, [pos], dl16, mask=m)
                plsc.store_scatter(s_list, [pos], sscans[b][sl], mask=m)
                plsc.store_scatter(v_list, [pos], vscans[b][sl], mask=m)
                return cntv + plsc.all_reduce_population_count(m)
            return lax.fori_loop(0, SCH // 16, scan_vec, cntv, unroll=5)

        scan_issue(0, 0)

        def scan_pair(ci2, cntv):
            ci = ci2 * 2

            @pl.when(ci + 1 < NSCAN)
            def _():
                scan_issue(ci + 1, 1)
            scan_drain(ci, 0)
            cntv = scan_compute(0, cntv)

            @pl.when(ci + 2 < NSCAN)
            def _():
                scan_issue(ci + 2, 0)
            scan_drain(ci + 1, 1)
            cntv = scan_compute(1, cntv)
            return cntv

        cntv = lax.fori_loop(0, NSCAN // 2, scan_pair,
                             jnp.zeros((16,), jnp.int32))

        for kk in range(2 * CH // 16):
            addr = jnp.minimum(cntv + iota + kk * 16, CAP - 1)
            zi = jnp.zeros((16,), jnp.int32)
            plsc.store_scatter(dl_list, [addr], zi, mask=None)
            plsc.store_scatter(s_list, [addr], zi, mask=None)
            plsc.store_scatter(v_list, [addr], zeros16, mask=None)

        cnt = jnp.max(cntv)
        nch = (cnt + CH - 1) // CH
        nch2 = 2 * ((nch + 1) // 2)
        lastb = jnp.maximum(nch2 - 1, 0) * CH

        def p2_issue(base, b):
            pltpu.async_copy(feat_hbm.at[s_list.at[pl.dslice(base, CH)]],
                             rowss[b], sems[b])

        def p2_drain(base, b):
            pltpu.make_async_copy(feat_hbm.at[s_list.at[pl.dslice(base, CH)]],
                                  rowss[b], sems[b]).wait()

        def p2_compute(base, b):
            cntb = jnp.full((16,), 0, jnp.int32) + (cnt - base)

            def edge_body(e, _):
                ev = jnp.full((16,), e, jnp.int32)
                vv = plsc.load_gather(v_list, [ev + base])
                vv = jnp.where(ev < cntb, vv, zeros16)
                dlv = plsc.load_gather(dl_list, [ev + base])
                for d in range(D // 16):
                    csl = pl.ds(d * 16, 16)
                    plsc.addupdate_scatter(
                        acc, [dlv, iota + d * 16], rowss[b][e, csl] * vv)
                return _
            lax.fori_loop(0, CH, edge_body, None)

        p2_issue(0, 0)

        def p2_pair(i2, _):
            base = i2 * 2 * CH
            p2_issue(jnp.minimum(base + CH, lastb), 1)
            p2_drain(base, 0)
            p2_compute(base, 0)
            p2_issue(jnp.minimum(base + 2 * CH, lastb), 0)
            p2_drain(jnp.minimum(base + CH, lastb), 1)
            p2_compute(base + CH, 1)
            return _
        lax.fori_loop(0, nch2 // 2, p2_pair, None)
        p2_drain(lastb, 0)

        pltpu.sync_copy(acc, out_hbm.at[pl.ds(lo, RANGE)])

    return k(features, src, dst, vals)


def _tc_combine(features, lap, W1, b1, W2, b2):
    BN = 1000
    bias = (b1 + b2).reshape(1, D)

    def body(f_ref, l_ref, w1_ref, w2_ref, b_ref, o_ref):
        lap_b = l_ref[...]
        f = f_ref[...]
        m1 = lap_b + f
        m2 = lap_b * f
        dn = (((1,), (1,)), ((), ()))
        o_ref[...] = (
            lax.dot_general(m1, w1_ref[...], dn,
                            preferred_element_type=jnp.float32)
            + lax.dot_general(m2, w2_ref[...], dn,
                              preferred_element_type=jnp.float32)
            + b_ref[...]
        )

    row_spec = pl.BlockSpec((BN, D), lambda i: (i, 0))
    full_spec = pl.BlockSpec((D, D), lambda i: (0, 0))
    return pl.pallas_call(
        body,
        grid=(N // BN,),
        in_specs=[row_spec, row_spec, full_spec, full_spec,
                  pl.BlockSpec((1, D), lambda i: (0, 0))],
        out_specs=row_spec,
        out_shape=jax.ShapeDtypeStruct((N, D), jnp.float32),
    )(features, lap, W1, W2, bias)


@jax.jit
def kernel(features, edge_index, edge_vals, W1, b1, W2, b2):
    dst = edge_index[0]
    src = edge_index[1]
    lap_pad = _sc_spmm(features, src, dst, edge_vals)
    return _tc_combine(features, lap_pad[:N], W1, b1, W2, b2)


# speedup vs baseline: 2.5848x; 2.1607x over previous
import functools

import jax
import jax.numpy as jnp
from jax import lax
from jax.experimental import pallas as pl
from jax.experimental.pallas import tpu as pltpu
from jax.experimental.pallas import tpu_sc as plsc

N = 10000
E = 320000
D = 128
NC = 2
NS = 16
NW = NC * NS
CH = 128
EPT = E // NW
NCHT = 80
EPTP = NCHT * CH
NACC = 10016
ZR = 16
RPS = 624
TAIL = NACC - NS * RPS


def _sc_spmm(features, src_p, dst_p, vals_p):
    mesh = plsc.VectorSubcoreMesh(core_axis_name="c", subcore_axis_name="s")

    @functools.partial(
        pl.kernel,
        out_type=jax.ShapeDtypeStruct((NC, NACC, D), jnp.float32),
        mesh=mesh,
        scratch_types=[
            pltpu.VMEM((EPTP,), jnp.int32),
            pltpu.VMEM((2, CH), jnp.int32),
            pltpu.VMEM((2, CH), jnp.float32),
            pltpu.VMEM((CH, D), jnp.float32),
            pltpu.VMEM((CH, D), jnp.float32),
            pltpu.VMEM((ZR, D), jnp.float32),
            pltpu.VMEM_SHARED((NACC, D), jnp.float32),
            pltpu.SemaphoreType.DMA,
            pltpu.SemaphoreType.DMA,
            pltpu.SemaphoreType.DMA,
            pltpu.SemaphoreType.DMA,
            pltpu.SemaphoreType.DMA,
            pltpu.SemaphoreType.DMA,
            pltpu.SemaphoreType.DMA,
        ],
        compiler_params=pltpu.CompilerParams(needs_layout_passes=False),
    )
    def k(feat_hbm, src_hbm, dst_hbm, vals_hbm, out_hbm,
          src_t, dst_b, val_b, rows0, rows1, zbuf, acc,
          g0, g1, s0, s1, d0, d1, zsem):
        c = lax.axis_index("c")
        s = lax.axis_index("s")
        wid = s * NC + c
        rows = (rows0, rows1)
        gsem = (g0, g1)
        ssem = (s0, s1)
        dsem = (d0, d1)

        def zero_row(r, _):
            for d in range(D // 16):
                zbuf[r, pl.ds(d * 16, 16)] = jnp.zeros((16,), jnp.float32)
            return _
        lax.fori_loop(0, ZR, zero_row, None)

        def zissue(j, _):
            pltpu.async_copy(zbuf, acc.at[pl.ds(s * RPS + j * ZR, ZR)],
                             zsem)
            return _
        lax.fori_loop(0, RPS // ZR, zissue, None)

        @pl.when(s == 0)
        def _():
            pltpu.async_copy(zbuf, acc.at[pl.ds(NS * RPS, ZR)], zsem)
            pltpu.async_copy(zbuf, acc.at[pl.ds(NS * RPS + ZR, ZR)], zsem)

        pltpu.sync_copy(src_hbm.at[pl.ds(wid * EPTP, EPTP)], src_t)

        def zdrain(j, _):
            pltpu.make_async_copy(
                zbuf, acc.at[pl.ds(s * RPS + j * ZR, ZR)], zsem).wait()
            return _
        lax.fori_loop(0, RPS // ZR, zdrain, None)

        @pl.when(s == 0)
        def _():
            pltpu.make_async_copy(zbuf, acc.at[pl.ds(NS * RPS, ZR)],
                                  zsem).wait()
            pltpu.make_async_copy(zbuf, acc.at[pl.ds(NS * RPS + ZR, ZR)],
                                  zsem).wait()
        plsc.subcore_barrier()

        def issue_gather(i, b):
            pltpu.async_copy(feat_hbm.at[src_t.at[pl.ds(i * CH, CH)]],
                             rows[b], gsem[b])

        def wait_gather(i, b):
            pltpu.make_async_copy(feat_hbm.at[src_t.at[pl.ds(i * CH, CH)]],
                                  rows[b], gsem[b]).wait()

        def issue_dv(i, b):
            pltpu.async_copy(dst_hbm.at[wid, i], dst_b.at[b], dsem[b])
            pltpu.async_copy(vals_hbm.at[wid, i], val_b.at[b], dsem[b])

        def wait_dv(i, b):
            pltpu.make_async_copy(dst_hbm.at[wid, i], dst_b.at[b],
                                  dsem[b]).wait()
            pltpu.make_async_copy(vals_hbm.at[wid, i], val_b.at[b],
                                  dsem[b]).wait()

        def issue_scatter(i, b):
            pltpu.async_copy(rows[b], acc.at[dst_b.at[b]], ssem[b],
                             add=True)

        def wait_scatter(i, b):
            pltpu.make_async_copy(rows[b], acc.at[dst_b.at[b]],
                                  ssem[b]).wait()

        def scale(i, b):
            def scale_edge(e, _):
                vv = plsc.load_gather(val_b.at[b],
                                      [jnp.full((16,), e, jnp.int32)])
                for d in range(D // 16):
                    sl = pl.ds(d * 16, 16)
                    rows[b][e, sl] = rows[b][e, sl] * vv
                return _
            lax.fori_loop(0, CH, scale_edge, None)

        issue_gather(0, 0)
        issue_dv(0, 0)
        wait_gather(0, 0)
        issue_gather(1, 1)
        issue_dv(1, 1)
        wait_dv(0, 0)
        scale(0, 0)
        issue_scatter(0, 0)

        def stepf(i, b):
            wait_gather(i, b)
            wait_scatter(i - 1, 1 - b)
            issue_gather(i + 1, 1 - b)
            issue_dv(i + 1, 1 - b)
            wait_dv(i, b)
            scale(i, b)
            issue_scatter(i, b)

        def pair(i2, _):
            i = 1 + 2 * i2
            stepf(i, 1)
            stepf(i + 1, 0)
            return _
        lax.fori_loop(0, (NCHT - 2) // 2, pair, None)

        wait_gather(NCHT - 1, 1)
        wait_scatter(NCHT - 2, 0)
        wait_dv(NCHT - 1, 1)
        scale(NCHT - 1, 1)
        issue_scatter(NCHT - 1, 1)
        wait_scatter(NCHT - 1, 1)

        plsc.subcore_barrier()

        def drain(j, _):
            off = s * RPS + j * ZR
            pltpu.async_copy(acc.at[pl.ds(off, ZR)],
                             out_hbm.at[c].at[pl.ds(off, ZR)], zsem)
            return _
        lax.fori_loop(0, RPS // ZR, drain, None)

        @pl.when(s == 0)
        def _():
            pltpu.async_copy(acc.at[pl.ds(NS * RPS, ZR)],
                             out_hbm.at[c].at[pl.ds(NS * RPS, ZR)], zsem)
            pltpu.async_copy(acc.at[pl.ds(NS * RPS + ZR, ZR)],
                             out_hbm.at[c].at[pl.ds(NS * RPS + ZR, ZR)],
                             zsem)

        def draind(j, _):
            off = s * RPS + j * ZR
            pltpu.make_async_copy(acc.at[pl.ds(off, ZR)],
                                  out_hbm.at[c].at[pl.ds(off, ZR)],
                                  zsem).wait()
            return _
        lax.fori_loop(0, RPS // ZR, draind, None)

        @pl.when(s == 0)
        def _():
            pltpu.make_async_copy(acc.at[pl.ds(NS * RPS, ZR)],
                                  out_hbm.at[c].at[pl.ds(NS * RPS, ZR)],
                                  zsem).wait()
            pltpu.make_async_copy(
                acc.at[pl.ds(NS * RPS + ZR, ZR)],
                out_hbm.at[c].at[pl.ds(NS * RPS + ZR, ZR)], zsem).wait()

    return k(features, src_p, dst_p, vals_p)


def _tc_combine(features, partial, W1, b1, W2, b2):
    BN = 1000
    bias = (b1 + b2).reshape(1, D)
    p0 = partial[0, :N]
    p1 = partial[1, :N]

    def body(f_ref, p0_ref, p1_ref, w1_ref, w2_ref, b_ref, o_ref):
        lap = p0_ref[...] + p1_ref[...]
        f = f_ref[...]
        m1 = lap + f
        m2 = lap * f
        dn = (((1,), (1,)), ((), ()))
        o_ref[...] = (
            lax.dot_general(m1, w1_ref[...], dn,
                            preferred_element_type=jnp.float32)
            + lax.dot_general(m2, w2_ref[...], dn,
                              preferred_element_type=jnp.float32)
            + b_ref[...]
        )

    row_spec = pl.BlockSpec((BN, D), lambda i: (i, 0))
    full_spec = pl.BlockSpec((D, D), lambda i: (0, 0))
    return pl.pallas_call(
        body,
        grid=(N // BN,),
        in_specs=[row_spec, row_spec, row_spec, full_spec, full_spec,
                  pl.BlockSpec((1, D), lambda i: (0, 0))],
        out_specs=row_spec,
        out_shape=jax.ShapeDtypeStruct((N, D), jnp.float32),
    )(features, p0, p1, W1, W2, bias)


@jax.jit
def kernel(features, edge_index, edge_vals, W1, b1, W2, b2):
    dst = edge_index[0]
    src = edge_index[1]
    src_pad = jnp.pad(src.reshape(NW, EPT),
                      ((0, 0), (0, EPTP - EPT))).reshape(NW * EPTP)
    dst_pad = jnp.pad(dst.reshape(NW, EPT), ((0, 0), (0, EPTP - EPT)),
                      constant_values=N).reshape(NW, NCHT, CH)
    vals_pad = jnp.pad(edge_vals.reshape(NW, EPT),
                       ((0, 0), (0, EPTP - EPT))).reshape(NW, NCHT, CH)
    partial = _sc_spmm(features, src_pad, dst_pad, vals_pad)
    return _tc_combine(features, partial, W1, b1, W2, b2)
